# baseline (speedup 1.0000x reference)
import functools

import jax
import jax.numpy as jnp
from jax import lax
from jax.experimental import pallas as pl
from jax.experimental.pallas import tpu as pltpu
from jax.experimental.pallas import tpu_sc as plsc

N = 10000
E = 320000
D = 128
H = 128
OUT = 128
EPS = 0.0
BN_EPS = 1e-5

NC = 2
NS = 16
NW = NC * NS
CH = 128
CPW = 160
E_PAD = NS * CPW * CH
N_PAD = 10240
RPT = N_PAD // NS

@functools.cache
def _sc_mesh():
    return plsc.VectorSubcoreMesh(core_axis_name="c", subcore_axis_name="s",
                                  num_cores=NC, num_subcores=NS)


def _aggr_body(h_hbm, src_hbm, dst_hbm, out_hbm,
               sidx0, sidx1, didx0, didx1, rows0, rows1,
               acc_sh, gsem0, gsem1, isem0, isem1):
    core = lax.axis_index("c")
    s = lax.axis_index("s")
    ebase = s * CPW * CH

    @pl.when(core == 0)
    def _core0_body():
        _aggr_core0(h_hbm, src_hbm, dst_hbm, out_hbm, sidx0, sidx1,
                    didx0, didx1, rows0, rows1, acc_sh,
                    gsem0, gsem1, isem0, isem1, s, ebase)


def _aggr_core0(h_hbm, src_hbm, dst_hbm, out_hbm, sidx0, sidx1,
                didx0, didx1, rows0, rows1, acc_sh,
                gsem0, gsem1, isem0, isem1, s, ebase):

    zeros16 = jnp.zeros((16,), jnp.float32)

    def _zero_buf(i, carry):
        r = i // (D // 16)
        col = (i % (D // 16)) * 16
        rows0[r, pl.ds(col, 16)] = zeros16
        return carry

    lax.fori_loop(0, CH * (D // 16), _zero_buf, 0)

    def _zero_acc(j, carry):
        pltpu.sync_copy(rows0, acc_sh.at[pl.ds(s * RPT + j * CH, CH)])
        return carry

    lax.fori_loop(0, RPT // CH, _zero_acc, 0)
    plsc.subcore_barrier()

    sidx = (sidx0, sidx1)
    didx = (didx0, didx1)
    rows = (rows0, rows1)
    gsem = (gsem0, gsem1)
    isem = (isem0, isem1)

    pltpu.sync_copy(src_hbm.at[pl.ds(ebase, CH)], sidx0)
    pltpu.sync_copy(dst_hbm.at[pl.ds(ebase, CH)], didx0)
    pltpu.async_copy(h_hbm.at[sidx0], rows0, gsem0)
    pltpu.async_copy(src_hbm.at[pl.ds(ebase + CH, CH)], sidx1, isem1)
    pltpu.async_copy(dst_hbm.at[pl.ds(ebase + CH, CH)], didx1, isem1)

    def _pair(i, carry):
        for b in range(2):
            c = 2 * i + b
            pltpu.make_async_copy(h_hbm.at[sidx[b]], rows[b], gsem[b]).wait()

            @pl.when(c + 1 < CPW)
            def _():
                pltpu.make_async_copy(
                    src_hbm.at[pl.ds(ebase, CH)], sidx[1 - b],
                    isem[1 - b]).wait()
                pltpu.make_async_copy(
                    dst_hbm.at[pl.ds(ebase, CH)], didx[1 - b],
                    isem[1 - b]).wait()
                pltpu.async_copy(h_hbm.at[sidx[1 - b]], rows[1 - b],
                                 gsem[1 - b])

            pltpu.sync_copy(rows[b], acc_sh.at[didx[b]], add=True)

            @pl.when(c + 2 < CPW)
            def _():
                nb = ebase + (c + 2) * CH
                pltpu.async_copy(src_hbm.at[pl.ds(nb, CH)], sidx[b], isem[b])
                pltpu.async_copy(dst_hbm.at[pl.ds(nb, CH)], didx[b], isem[b])
        return carry

    lax.fori_loop(0, CPW // 2, _pair, 0)
    plsc.subcore_barrier()

    pltpu.sync_copy(acc_sh.at[pl.ds(s * RPT, RPT)],
                    out_hbm.at[pl.ds(s * RPT, RPT)])


@functools.cache
def _aggr():
    return pl.kernel(
        _aggr_body,
        out_type=jax.ShapeDtypeStruct((N_PAD, D), jnp.float32),
        mesh=_sc_mesh(),
        scratch_types=[
            pltpu.VMEM((CH,), jnp.int32),
            pltpu.VMEM((CH,), jnp.int32),
            pltpu.VMEM((CH,), jnp.int32),
            pltpu.VMEM((CH,), jnp.int32),
            pltpu.VMEM((CH, D), jnp.float32),
            pltpu.VMEM((CH, D), jnp.float32),
            pltpu.VMEM_SHARED((N_PAD, D), jnp.float32),
            pltpu.SemaphoreType.DMA,
            pltpu.SemaphoreType.DMA,
            pltpu.SemaphoreType.DMA,
            pltpu.SemaphoreType.DMA,
        ],
    )


def _mlp_block(z, W1, b1, g, beta, W2, b2):
    y = jnp.dot(z, W1, preferred_element_type=jnp.float32) + b1
    mu = jnp.mean(y, axis=0, keepdims=True)
    var = jnp.mean((y - mu) ** 2, axis=0, keepdims=True)
    y = (y - mu) / jnp.sqrt(var + BN_EPS) * g + beta
    y = jnp.maximum(y, 0.0)
    return jnp.dot(y, W2, preferred_element_type=jnp.float32) + b2


def _layer_kernel(h_ref, p_ref, W1_ref, b1_ref, g_ref, beta_ref,
                  W2_ref, b2_ref, o_ref):
    h = h_ref[pl.ds(0, N), :]
    z = (1.0 + EPS) * h + p_ref[pl.ds(0, N), :]
    out = _mlp_block(z, W1_ref[...], b1_ref[...], g_ref[...], beta_ref[...],
                     W2_ref[...], b2_ref[...])
    o_ref[pl.ds(0, N), :] = jnp.maximum(out, 0.0)
    o_ref[pl.ds(N, N_PAD - N), :] = jnp.zeros((N_PAD - N, D), jnp.float32)


def _final_kernel(h_ref, p_ref, W1_ref, b1_ref, g_ref, beta_ref,
                  W2_ref, b2_ref, Wc1_ref, bc1_ref, Wc2_ref, bc2_ref, o_ref):
    h = h_ref[pl.ds(0, N), :]
    z = (1.0 + EPS) * h + p_ref[pl.ds(0, N), :]
    out = _mlp_block(z, W1_ref[...], b1_ref[...], g_ref[...], beta_ref[...],
                     W2_ref[...], b2_ref[...])
    h3 = jnp.maximum(out, 0.0)
    hc = jnp.maximum(
        jnp.dot(h3, Wc1_ref[...], preferred_element_type=jnp.float32)
        + bc1_ref[...], 0.0)
    o_ref[...] = (jnp.dot(hc, Wc2_ref[...], preferred_element_type=jnp.float32)
                  + bc2_ref[...])


_layer = pl.pallas_call(
    _layer_kernel,
    out_shape=jax.ShapeDtypeStruct((N_PAD, D), jnp.float32),
)

_final = pl.pallas_call(
    _final_kernel,
    out_shape=jax.ShapeDtypeStruct((N, OUT), jnp.float32),
)


def kernel(x, edge_index, W0_1, b0_1, g0, beta0, W0_2, b0_2,
           W1_1, b1_1, g1, beta1, W1_2, b1_2,
           W2_1, b2_1, g2, beta2, W2_2, b2_2, Wc1, bc1, Wc2, bc2):
    src = edge_index[0]
    dst = edge_index[1]
    pad = jnp.full((E_PAD - E,), N, dtype=jnp.int32)
    src_p = jnp.concatenate([src, pad])
    dst_p = jnp.concatenate([dst, pad])

    h = jnp.zeros((N_PAD, D), jnp.float32).at[:N].set(x)

    params = [
        (W0_1, b0_1, g0, beta0, W0_2, b0_2),
        (W1_1, b1_1, g1, beta1, W1_2, b1_2),
        (W2_1, b2_1, g2, beta2, W2_2, b2_2),
    ]

    def row(v):
        return v.reshape(1, -1)

    aggr = _aggr()
    for i in range(2):
        W1, b1, g, beta, W2, b2 = params[i]
        partials = aggr(h, src_p, dst_p)
        h = _layer(h, partials, W1, row(b1), row(g), row(beta), W2, row(b2))

    W1, b1, g, beta, W2, b2 = params[2]
    partials = aggr(h, src_p, dst_p)
    return _final(h, partials, W1, row(b1), row(g), row(beta), W2, row(b2),
                  Wc1, row(bc1), Wc2, row(bc2))


# speedup vs baseline: 2.4517x; 2.4517x over previous
import functools

import jax
import jax.numpy as jnp
from jax import lax
from jax.experimental import pallas as pl
from jax.experimental.pallas import tpu as pltpu
from jax.experimental.pallas import tpu_sc as plsc

N = 10000
E = 320000
D = 128
H = 128
OUT = 128
EPS = 0.0
BN_EPS = 1e-5

NC = 2
NS = 16
NW = NC * NS
CH = 128
CPW = 80
E_PAD = NW * CPW * CH
N_PAD = 10240
RPT = N_PAD // NS
DH = D // 2


@functools.cache
def _sc_mesh():
    return plsc.VectorSubcoreMesh(core_axis_name="c", subcore_axis_name="s",
                                  num_cores=NC, num_subcores=NS)


def _aggr_body(h_hbm, src_hbm, dst_hbm, out_hbm,
               sidx, didx, rows0, rows1, h_sh, acc_sh, gsem0, gsem1):
    core = lax.axis_index("c")
    s = lax.axis_index("s")
    wid = core * NS + s

    pltpu.sync_copy(src_hbm.at[pl.ds(wid * CPW, CPW)], sidx)
    pltpu.sync_copy(dst_hbm.at[pl.ds(wid * CPW, CPW)], didx)

    rows = (rows0, rows1)
    gsem = (gsem0, gsem1)
    zeros16 = jnp.zeros((16,), jnp.float32)

    for half in range(2):
        pltpu.sync_copy(h_hbm.at[half, pl.ds(s * RPT, RPT)],
                        h_sh.at[pl.ds(s * RPT, RPT)])

        def _zb(i, carry):
            r = i // (DH // 16)
            col = (i % (DH // 16)) * 16
            rows0[r, pl.ds(col, 16)] = zeros16
            return carry

        lax.fori_loop(0, CH * (DH // 16), _zb, 0)

        def _za(j, carry):
            pltpu.sync_copy(rows0, acc_sh.at[pl.ds(s * RPT + j * CH, CH)])
            return carry

        lax.fori_loop(0, RPT // CH, _za, 0)
        plsc.subcore_barrier()

        pltpu.async_copy(h_sh.at[sidx.at[0]], rows0, gsem0)

        def _pair(i, carry):
            for b in range(2):
                c = 2 * i + b
                pltpu.make_async_copy(h_sh.at[sidx.at[c]],
                                      rows[b], gsem[b]).wait()

                @pl.when(c + 1 < CPW)
                def _():
                    pltpu.async_copy(h_sh.at[sidx.at[c + 1]],
                                     rows[1 - b], gsem[1 - b])

                pltpu.sync_copy(rows[b], acc_sh.at[didx.at[c]], add=True)
            return carry

        lax.fori_loop(0, CPW // 2, _pair, 0)
        plsc.subcore_barrier()

        pltpu.sync_copy(acc_sh.at[pl.ds(s * RPT, RPT)],
                        out_hbm.at[core, half, pl.ds(s * RPT, RPT)])


@functools.cache
def _aggr():
    return pl.kernel(
        _aggr_body,
        out_type=jax.ShapeDtypeStruct((NC, 2, N_PAD, DH), jnp.float32),
        mesh=_sc_mesh(),
        compiler_params=pltpu.CompilerParams(use_tc_tiling_on_sc=False),
        scratch_types=[
            pltpu.VMEM((CPW, CH), jnp.int32),
            pltpu.VMEM((CPW, CH), jnp.int32),
            pltpu.VMEM((CH, DH), jnp.float32),
            pltpu.VMEM((CH, DH), jnp.float32),
            pltpu.VMEM_SHARED((N_PAD, DH), jnp.float32),
            pltpu.VMEM_SHARED((N_PAD, DH), jnp.float32),
            pltpu.SemaphoreType.DMA,
            pltpu.SemaphoreType.DMA,
        ],
    )


def _mlp_block(z, W1, b1, g, beta, W2, b2):
    y = jnp.dot(z, W1, preferred_element_type=jnp.float32) + b1
    mu = jnp.mean(y, axis=0, keepdims=True)
    var = jnp.mean((y - mu) ** 2, axis=0, keepdims=True)
    y = (y - mu) / jnp.sqrt(var + BN_EPS) * g + beta
    y = jnp.maximum(y, 0.0)
    return jnp.dot(y, W2, preferred_element_type=jnp.float32) + b2


def _gin_input(h_ref, p_ref):
    h = jnp.concatenate(
        [h_ref[0, pl.ds(0, N), :], h_ref[1, pl.ds(0, N), :]], axis=1)
    a0 = p_ref[0, 0, pl.ds(0, N), :] + p_ref[1, 0, pl.ds(0, N), :]
    a1 = p_ref[0, 1, pl.ds(0, N), :] + p_ref[1, 1, pl.ds(0, N), :]
    return (1.0 + EPS) * h + jnp.concatenate([a0, a1], axis=1)


def _layer_kernel(h_ref, p_ref, W1_ref, b1_ref, g_ref, beta_ref,
                  W2_ref, b2_ref, o_ref):
    z = _gin_input(h_ref, p_ref)
    out = _mlp_block(z, W1_ref[...], b1_ref[...], g_ref[...], beta_ref[...],
                     W2_ref[...], b2_ref[...])
    h2 = jnp.maximum(out, 0.0)
    zpad = jnp.zeros((N_PAD - N, DH), jnp.float32)
    o_ref[0, pl.ds(0, N), :] = h2[:, :DH]
    o_ref[1, pl.ds(0, N), :] = h2[:, DH:]
    o_ref[0, pl.ds(N, N_PAD - N), :] = zpad
    o_ref[1, pl.ds(N, N_PAD - N), :] = zpad


def _final_kernel(h_ref, p_ref, W1_ref, b1_ref, g_ref, beta_ref,
                  W2_ref, b2_ref, Wc1_ref, bc1_ref, Wc2_ref, bc2_ref, o_ref):
    z = _gin_input(h_ref, p_ref)
    out = _mlp_block(z, W1_ref[...], b1_ref[...], g_ref[...], beta_ref[...],
                     W2_ref[...], b2_ref[...])
    h3 = jnp.maximum(out, 0.0)
    hc = jnp.maximum(
        jnp.dot(h3, Wc1_ref[...], preferred_element_type=jnp.float32)
        + bc1_ref[...], 0.0)
    o_ref[...] = (jnp.dot(hc, Wc2_ref[...], preferred_element_type=jnp.float32)
                  + bc2_ref[...])


_layer = pl.pallas_call(
    _layer_kernel,
    out_shape=jax.ShapeDtypeStruct((2, N_PAD, DH), jnp.float32),
)

_final = pl.pallas_call(
    _final_kernel,
    out_shape=jax.ShapeDtypeStruct((N, OUT), jnp.float32),
)


def kernel(x, edge_index, W0_1, b0_1, g0, beta0, W0_2, b0_2,
           W1_1, b1_1, g1, beta1, W1_2, b1_2,
           W2_1, b2_1, g2, beta2, W2_2, b2_2, Wc1, bc1, Wc2, bc2):
    src = edge_index[0]
    dst = edge_index[1]
    pad = jnp.full((E_PAD - E,), N, dtype=jnp.int32)
    src_p = jnp.concatenate([src, pad]).reshape(NW * CPW, CH)
    dst_p = jnp.concatenate([dst, pad]).reshape(NW * CPW, CH)

    h = (jnp.zeros((2, N_PAD, DH), jnp.float32)
         .at[0, :N].set(x[:, :DH]).at[1, :N].set(x[:, DH:]))

    params = [
        (W0_1, b0_1, g0, beta0, W0_2, b0_2),
        (W1_1, b1_1, g1, beta1, W1_2, b1_2),
        (W2_1, b2_1, g2, beta2, W2_2, b2_2),
    ]

    def row(v):
        return v.reshape(1, -1)

    aggr = _aggr()
    for i in range(2):
        W1, b1, g, beta, W2, b2 = params[i]
        partials = aggr(h, src_p, dst_p)
        h = _layer(h, partials, W1, row(b1), row(g), row(beta), W2, row(b2))

    W1, b1, g, beta, W2, b2 = params[2]
    partials = aggr(h, src_p, dst_p)
    return _final(h, partials, W1, row(b1), row(g), row(beta), W2, row(b2),
                  Wc1, row(bc1), Wc2, row(bc2))


# speedup vs baseline: 2.7287x; 1.1130x over previous
import functools

import jax
import jax.numpy as jnp
from jax import lax
from jax.experimental import pallas as pl
from jax.experimental.pallas import tpu as pltpu
from jax.experimental.pallas import tpu_sc as plsc

N = 10000
E = 320000
D = 128
H = 128
OUT = 128
EPS = 0.0
BN_EPS = 1e-5

NC = 2
NS = 16
NW = NC * NS
CH = 128
CPW = 81
E_PAD = NW * CPW * CH
N_PAD = 10240
RPT = N_PAD // NS
DH = D // 2


@functools.cache
def _sc_mesh():
    return plsc.VectorSubcoreMesh(core_axis_name="c", subcore_axis_name="s",
                                  num_cores=NC, num_subcores=NS)


def _aggr_body(h_hbm, src_hbm, dst_hbm, out_hbm,
               sidx, didx, rows0, rows1, rows2, h_sh, acc_sh,
               gsem0, gsem1, gsem2, ssem0, ssem1, ssem2):
    core = lax.axis_index("c")
    s = lax.axis_index("s")
    wid = core * NS + s

    pltpu.sync_copy(src_hbm.at[pl.ds(wid * CPW, CPW)], sidx)
    pltpu.sync_copy(dst_hbm.at[pl.ds(wid * CPW, CPW)], didx)

    rows = (rows0, rows1, rows2)
    gsem = (gsem0, gsem1, gsem2)
    ssem = (ssem0, ssem1, ssem2)
    zeros16 = jnp.zeros((16,), jnp.float32)

    for half in range(2):
        pltpu.sync_copy(h_hbm.at[half, pl.ds(s * RPT, RPT)],
                        h_sh.at[pl.ds(s * RPT, RPT)])

        def _zb(i, carry):
            r = i // (DH // 16)
            col = (i % (DH // 16)) * 16
            rows0[r, pl.ds(col, 16)] = zeros16
            return carry

        lax.fori_loop(0, CH * (DH // 16), _zb, 0)

        def _za(j, carry):
            pltpu.sync_copy(rows0, acc_sh.at[pl.ds(s * RPT + j * CH, CH)])
            return carry

        lax.fori_loop(0, RPT // CH, _za, 0)
        plsc.subcore_barrier()

        pltpu.async_copy(h_sh.at[sidx.at[0]], rows0, gsem0)
        pltpu.async_copy(h_sh.at[sidx.at[1]], rows1, gsem1)

        def _trip(i, carry):
            for b in range(3):
                c = 3 * i + b
                pltpu.make_async_copy(h_sh.at[sidx.at[c]],
                                      rows[b], gsem[b]).wait()
                pltpu.async_copy(rows[b], acc_sh.at[didx.at[c]],
                                 ssem[b], add=True)
                nb = (b + 2) % 3

                @pl.when((c >= 1) & (c + 2 < CPW))
                def _():
                    pltpu.make_async_copy(rows[nb], acc_sh.at[didx.at[c]],
                                          ssem[nb]).wait()

                @pl.when(c + 2 < CPW)
                def _():
                    pltpu.async_copy(h_sh.at[sidx.at[c + 2]],
                                     rows[nb], gsem[nb])
            return carry

        lax.fori_loop(0, CPW // 3, _trip, 0)
        for tail in range(3):
            b = (CPW - 3 + tail) % 3
            pltpu.make_async_copy(rows[b], acc_sh.at[didx.at[0]],
                                  ssem[b]).wait()
        plsc.subcore_barrier()

        pltpu.sync_copy(acc_sh.at[pl.ds(s * RPT, RPT)],
                        out_hbm.at[core, half, pl.ds(s * RPT, RPT)])


@functools.cache
def _aggr():
    return pl.kernel(
        _aggr_body,
        out_type=jax.ShapeDtypeStruct((NC, 2, N_PAD, DH), jnp.float32),
        mesh=_sc_mesh(),
        compiler_params=pltpu.CompilerParams(use_tc_tiling_on_sc=False),
        scratch_types=[
            pltpu.VMEM((CPW, CH), jnp.int32),
            pltpu.VMEM((CPW, CH), jnp.int32),
            pltpu.VMEM((CH, DH), jnp.float32),
            pltpu.VMEM((CH, DH), jnp.float32),
            pltpu.VMEM((CH, DH), jnp.float32),
            pltpu.VMEM_SHARED((N_PAD, DH), jnp.float32),
            pltpu.VMEM_SHARED((N_PAD, DH), jnp.float32),
            pltpu.SemaphoreType.DMA,
            pltpu.SemaphoreType.DMA,
            pltpu.SemaphoreType.DMA,
            pltpu.SemaphoreType.DMA,
            pltpu.SemaphoreType.DMA,
            pltpu.SemaphoreType.DMA,
        ],
    )


def _mlp_block(z, W1, b1, g, beta, W2, b2):
    y = jnp.dot(z, W1, preferred_element_type=jnp.float32) + b1
    mu = jnp.mean(y, axis=0, keepdims=True)
    var = jnp.mean((y - mu) ** 2, axis=0, keepdims=True)
    y = (y - mu) / jnp.sqrt(var + BN_EPS) * g + beta
    y = jnp.maximum(y, 0.0)
    return jnp.dot(y, W2, preferred_element_type=jnp.float32) + b2


def _gin_input(h_ref, p_ref):
    h = jnp.concatenate(
        [h_ref[0, pl.ds(0, N), :], h_ref[1, pl.ds(0, N), :]], axis=1)
    a0 = p_ref[0, 0, pl.ds(0, N), :] + p_ref[1, 0, pl.ds(0, N), :]
    a1 = p_ref[0, 1, pl.ds(0, N), :] + p_ref[1, 1, pl.ds(0, N), :]
    return (1.0 + EPS) * h + jnp.concatenate([a0, a1], axis=1)


def _layer_kernel(h_ref, p_ref, W1_ref, b1_ref, g_ref, beta_ref,
                  W2_ref, b2_ref, o_ref):
    z = _gin_input(h_ref, p_ref)
    out = _mlp_block(z, W1_ref[...], b1_ref[...], g_ref[...], beta_ref[...],
                     W2_ref[...], b2_ref[...])
    h2 = jnp.maximum(out, 0.0)
    zpad = jnp.zeros((N_PAD - N, DH), jnp.float32)
    o_ref[0, pl.ds(0, N), :] = h2[:, :DH]
    o_ref[1, pl.ds(0, N), :] = h2[:, DH:]
    o_ref[0, pl.ds(N, N_PAD - N), :] = zpad
    o_ref[1, pl.ds(N, N_PAD - N), :] = zpad


def _final_kernel(h_ref, p_ref, W1_ref, b1_ref, g_ref, beta_ref,
                  W2_ref, b2_ref, Wc1_ref, bc1_ref, Wc2_ref, bc2_ref, o_ref):
    z = _gin_input(h_ref, p_ref)
    out = _mlp_block(z, W1_ref[...], b1_ref[...], g_ref[...], beta_ref[...],
                     W2_ref[...], b2_ref[...])
    h3 = jnp.maximum(out, 0.0)
    hc = jnp.maximum(
        jnp.dot(h3, Wc1_ref[...], preferred_element_type=jnp.float32)
        + bc1_ref[...], 0.0)
    o_ref[...] = (jnp.dot(hc, Wc2_ref[...], preferred_element_type=jnp.float32)
                  + bc2_ref[...])


_layer = pl.pallas_call(
    _layer_kernel,
    out_shape=jax.ShapeDtypeStruct((2, N_PAD, DH), jnp.float32),
)

_final = pl.pallas_call(
    _final_kernel,
    out_shape=jax.ShapeDtypeStruct((N, OUT), jnp.float32),
)


def kernel(x, edge_index, W0_1, b0_1, g0, beta0, W0_2, b0_2,
           W1_1, b1_1, g1, beta1, W1_2, b1_2,
           W2_1, b2_1, g2, beta2, W2_2, b2_2, Wc1, bc1, Wc2, bc2):
    src = edge_index[0]
    dst = edge_index[1]
    pad = jnp.full((E_PAD - E,), N, dtype=jnp.int32)
    src_p = jnp.concatenate([src, pad]).reshape(NW * CPW, CH)
    dst_p = jnp.concatenate([dst, pad]).reshape(NW * CPW, CH)

    h = (jnp.zeros((2, N_PAD, DH), jnp.float32)
         .at[0, :N].set(x[:, :DH]).at[1, :N].set(x[:, DH:]))

    params = [
        (W0_1, b0_1, g0, beta0, W0_2, b0_2),
        (W1_1, b1_1, g1, beta1, W1_2, b1_2),
        (W2_1, b2_1, g2, beta2, W2_2, b2_2),
    ]

    def row(v):
        return v.reshape(1, -1)

    aggr = _aggr()
    for i in range(2):
        W1, b1, g, beta, W2, b2 = params[i]
        partials = aggr(h, src_p, dst_p)
        h = _layer(h, partials, W1, row(b1), row(g), row(beta), W2, row(b2))

    W1, b1, g, beta, W2, b2 = params[2]
    partials = aggr(h, src_p, dst_p)
    return _final(h, partials, W1, row(b1), row(g), row(beta), W2, row(b2),
                  Wc1, row(bc1), Wc2, row(bc2))
